# Initial kernel scaffold; baseline (speedup 1.0000x reference)
import jax
import jax.numpy as jnp
from jax.experimental import pallas as pl


def kernel(variable_prediction, label, graph_map, batch_variable_map, batch_function_map, edge_feature, meta_data, global_step, eps):
    raise NotImplementedError("write your pallas kernel here")


# speedup vs baseline: 25.8197x; 25.8197x over previous
import functools

import jax
import jax.numpy as jnp
from jax import lax
from jax.experimental import pallas as pl
from jax.experimental.pallas import tpu as pltpu
from jax.experimental.pallas import tpu_sc as plsc

_ALPHA = 0.4
_MAX_COEFF = 10.0

_NC = 2
_NS = 16
_NW = _NC * _NS
_L = 16

_CHUNK = 128


def _sc_edge_kernel(V, F_pad, E_pad):
    epw = E_pad // _NW
    n_chunks = epw // _CHUNK
    fpt = F_pad // _NS

    mesh = plsc.VectorSubcoreMesh(core_axis_name="c", subcore_axis_name="s",
                                  num_cores=_NC, num_subcores=_NS)

    @functools.partial(
        pl.kernel,
        out_type=(
            jax.ShapeDtypeStruct((_NC, F_pad), jnp.float32),
            jax.ShapeDtypeStruct((_NC, F_pad), jnp.float32),
        ),
        mesh=mesh,
        compiler_params=pltpu.CompilerParams(needs_layout_passes=False),
        scratch_types=dict(
            vp_v=pltpu.VMEM((V,), jnp.float32),
            vidx_v=pltpu.VMEM((_CHUNK,), jnp.int32),
            fidx_v=pltpu.VMEM((_CHUNK,), jnp.int32),
            ef_v=pltpu.VMEM((_CHUNK,), jnp.float32),
            wev_v=pltpu.VMEM((_CHUNK,), jnp.float32),
            w_v=pltpu.VMEM((_CHUNK,), jnp.float32),
            coeff_v=pltpu.VMEM((_L,), jnp.float32),
            nom_s=pltpu.VMEM_SHARED((F_pad,), jnp.float32),
            den_s=pltpu.VMEM_SHARED((F_pad,), jnp.float32),
        ),
    )
    def body(vp_hbm, vidx_hbm, fidx_hbm, ef_hbm, coeff_hbm, zeros_hbm,
             nom_out, den_out,
             vp_v, vidx_v, fidx_v, ef_v, wev_v, w_v, coeff_v, nom_s, den_s):
        cid = lax.axis_index("c")
        sid = lax.axis_index("s")
        wid = sid * _NC + cid
        base = wid * epw

        pltpu.sync_copy(vp_hbm, vp_v)
        pltpu.sync_copy(coeff_hbm, coeff_v)
        pltpu.sync_copy(zeros_hbm.at[pl.ds(sid * fpt, fpt)],
                        nom_s.at[pl.ds(sid * fpt, fpt)])
        pltpu.sync_copy(zeros_hbm.at[pl.ds(sid * fpt, fpt)],
                        den_s.at[pl.ds(sid * fpt, fpt)])
        plsc.subcore_barrier()

        coeff = coeff_v[...]

        def chunk_body(i, _):
            off = base + i * _CHUNK
            pltpu.sync_copy(vidx_hbm.at[pl.ds(off, _CHUNK)], vidx_v)
            pltpu.sync_copy(fidx_hbm.at[pl.ds(off, _CHUNK)], fidx_v)
            pltpu.sync_copy(ef_hbm.at[pl.ds(off, _CHUNK)], ef_v)
            for j in range(_CHUNK // _L):
                sl = pl.ds(j * _L, _L)
                idx = vidx_v[sl]
                vpred = plsc.load_gather(vp_v, [idx])
                ef = ef_v[sl]
                ev = ef * vpred + (1.0 - ef) * 0.5
                w = jnp.exp(coeff * ev)
                wev_v[sl] = w * ev
                w_v[sl] = w
            pltpu.sync_copy(wev_v, nom_s.at[fidx_v], add=True)
            pltpu.sync_copy(w_v, den_s.at[fidx_v], add=True)
            return ()

        lax.fori_loop(0, n_chunks, chunk_body, ())
        plsc.subcore_barrier()

        pltpu.sync_copy(nom_s.at[pl.ds(sid * fpt, fpt)],
                        nom_out.at[cid, pl.ds(sid * fpt, fpt)])
        pltpu.sync_copy(den_s.at[pl.ds(sid * fpt, fpt)],
                        den_out.at[cid, pl.ds(sid * fpt, fpt)])

    return body


def _tc_finalize_kernel(eps_ref, nom_ref, den_ref, out_ref, *, F, rows):
    nom = nom_ref[0] + nom_ref[1]
    den = den_ref[0] + den_ref[1]
    eps = eps_ref[0, 0]
    cv = den / jnp.maximum(nom, eps)
    d = cv - 1.0
    d2 = d * d
    cv = 1.0 + d2 * d2 * d
    lg = jnp.log(jnp.maximum(cv, eps))
    ridx = lax.broadcasted_iota(jnp.int32, (rows, 128), 0)
    cidx = lax.broadcasted_iota(jnp.int32, (rows, 128), 1)
    mask = (ridx * 128 + cidx) < F
    out_ref[...] = jnp.reshape(jnp.sum(jnp.where(mask, lg, 0.0)) / F, (1, 1))


def kernel(variable_prediction, label, graph_map, batch_variable_map,
           batch_function_map, edge_feature, meta_data, global_step, eps):
    del label, batch_variable_map, meta_data
    V = variable_prediction.shape[0]
    F = batch_function_map.shape[0]
    E = graph_map.shape[1]

    rows = (F + 128 * _NS - 1) // (128 * _NS) * _NS
    F_pad = rows * 128
    eg = _NW * _CHUNK
    E_pad = (E + eg - 1) // eg * eg

    coeff = jnp.minimum(jnp.power(global_step[0], _ALPHA),
                        jnp.float32(_MAX_COEFF))
    coeff16 = jnp.full((_L,), coeff, dtype=jnp.float32)

    vp_flat = variable_prediction.reshape(V)
    pad_e = E_pad - E
    var_idx = jnp.pad(graph_map[0], (0, pad_e))
    fun_idx = jnp.pad(graph_map[1], (0, pad_e), constant_values=F_pad - 1)
    ef_flat = jnp.pad(edge_feature.reshape(E), (0, pad_e))
    zeros = jnp.zeros((F_pad,), jnp.float32)

    nom_p, den_p = _sc_edge_kernel(V, F_pad, E_pad)(
        vp_flat, var_idx, fun_idx, ef_flat, coeff16, zeros)

    nom_p = nom_p.reshape(_NC, rows, 128)
    den_p = den_p.reshape(_NC, rows, 128)
    out = pl.pallas_call(
        functools.partial(_tc_finalize_kernel, F=F, rows=rows),
        out_shape=jax.ShapeDtypeStruct((1, 1), jnp.float32),
        in_specs=[
            pl.BlockSpec(memory_space=pltpu.SMEM),
            pl.BlockSpec((_NC, rows, 128), lambda: (0, 0, 0)),
            pl.BlockSpec((_NC, rows, 128), lambda: (0, 0, 0)),
        ],
        out_specs=pl.BlockSpec((1, 1), lambda: (0, 0)),
    )(eps.reshape(1, 1), nom_p, den_p)
    return out.reshape(())
